# baseline (speedup 1.0000x reference)
import functools

import jax
import jax.numpy as jnp
from jax import lax
from jax.experimental import pallas as pl
from jax.experimental.pallas import tpu as pltpu
from jax.experimental.pallas import tpu_sc as plsc

N = 10000
E = 160000
D = 256
H = 128
B = 4096

NC = 2
NS = 16
NW = NC * NS
CH = 128
NCHUNK = E // CH
NJ = NCHUNK // NW
REM = NCHUNK % NW
NPAD = 10240
POS_PER_TILE = NPAD // NS
BP = 6144
DUMP = B
DSPREAD = 2048
APT = BP // NS
BPT = B // NS
MMB = 400
CB = 256


def _mm_body(x_ref, w_ref, z1_ref, y2_ref):
    zy = jnp.dot(x_ref[...], w_ref[...], preferred_element_type=jnp.float32)
    z1_ref[...] = zy[:, :H]
    y2_ref[...] = zy[:, H:]


def _combine_body(zb_ref, a0_ref, a1_ref, d0_ref, d1_ref, o_ref):
    deg = jnp.maximum(d0_ref[...] + d1_ref[...], 1.0)
    mean = (a0_ref[...] + a1_ref[...]) / deg[:, None]
    o_ref[...] = jnp.maximum(zb_ref[...] + mean, 0.0)


def _sc_body(src_hbm, dst_hbm, nodes_hbm, y2_hbm, z1_hbm,
             aggb_hbm, degb_hbm, zb_hbm,
             idx_v, dst_v, val_v, pos_v, rows_v, rows2_v, rows3_v,
             ones_v, zdeg_v, deg1_v, sbuf_v, dtmp_v, dbuf_v,
             agg_sp, deg_sp, pos_sp, sem, sg0, sg1, sg2, ss0, ss1, ss2):
    c = lax.axis_index("c")
    s = lax.axis_index("s")
    w = c * NS + s

    lane = lax.iota(jnp.int32, 16)
    zero16 = jnp.zeros((16,), jnp.float32)
    ones16 = jnp.ones((16,), jnp.float32)
    dump16 = jnp.full((16,), DUMP, jnp.int32)

    def fill_rows(i, carry):
        for j in range(H // 16):
            rows_v[i, pl.ds(j * 16, 16)] = zero16
        return carry

    lax.fori_loop(0, CH, fill_rows, 0)
    for j in range(CH // 16):
        ones_v[pl.ds(j * 16, 16)] = ones16
        zdeg_v[pl.ds(j * 16, 16)] = zero16

    for m in range(POS_PER_TILE // CH):
        pbase = s * POS_PER_TILE + m * CH
        for j in range(CH // 16):
            g16 = pbase + j * 16 + lane
            val_v[pl.ds(j * 16, 16)] = DUMP + (g16 & (DSPREAD - 1))
        pltpu.sync_copy(val_v, pos_sp.at[pl.ds(pbase, CH)])
    zbase = s * APT
    for m in range(APT // CH):
        pltpu.sync_copy(rows_v, agg_sp.at[pl.ds(zbase + m * CH, CH)])
        pltpu.sync_copy(zdeg_v, deg_sp.at[pl.ds(zbase + m * CH, CH)])
    plsc.subcore_barrier()

    def pos_scatter(h, carry):
        nbase = s * BPT + h * CH
        pltpu.sync_copy(nodes_hbm.at[pl.ds(nbase, CH)], idx_v)
        for k in range(CH // 16):
            val_v[pl.ds(k * 16, 16)] = nbase + k * 16 + lane
        pltpu.sync_copy(val_v, pos_sp.at[idx_v])
        return carry

    lax.fori_loop(0, BPT // CH, pos_scatter, 0)
    plsc.subcore_barrier()

    pltpu.sync_copy(pos_sp, pos_v)

    cb = (w * NJ + jnp.minimum(w, REM)) * CH
    pltpu.sync_copy(src_hbm.at[pl.ds(cb, NJ * CH)], sbuf_v)
    pltpu.sync_copy(dst_hbm.at[pl.ds(cb, NJ * CH)], dtmp_v)

    def gather_rows(la, rbuf, sg):
        pltpu.async_copy(y2_hbm.at[sbuf_v.at[pl.ds(la * CH, CH)]], rbuf, sg)

    def wait_gather(rbuf, sg):
        pltpu.make_async_copy(y2_hbm.at[sbuf_v.at[pl.ds(0, CH)]], rbuf, sg).wait()

    def scatter_rows(la, rbuf, ss):
        pltpu.async_copy(rbuf, agg_sp.at[dbuf_v.at[la]], ss, add=True)
        pltpu.async_copy(ones_v, deg_sp.at[dbuf_v.at[la]], ss, add=True)

    def wait_scatter(rbuf, ss):
        pltpu.make_async_copy(rbuf, agg_sp.at[dbuf_v.at[0]], ss).wait()
        pltpu.make_async_copy(ones_v, deg_sp.at[dbuf_v.at[0]], ss).wait()

    def trans_body(j, carry):
        for k in range(CH // 16):
            d16 = dtmp_v[pl.ds(j * CH + k * 16, 16)]
            dbuf_v[j, pl.ds(k * 16, 16)] = plsc.load_gather(pos_v, [d16])
        return carry

    lax.fori_loop(0, NJ, trans_body, 0)

    slots = [(rows_v, sg0, ss0), (rows2_v, sg1, ss1), (rows3_v, sg2, ss2)]
    NSL = len(slots)
    NR = NJ // NSL

    def edge_round(r, carry):
        for t, (rb, sg, ss) in enumerate(slots):
            @pl.when(r > 0)
            def _(rb=rb, ss=ss):
                wait_scatter(rb, ss)

            gather_rows(NSL * r + t, rb, sg)
        for t, (rb, sg, ss) in enumerate(slots):
            wait_gather(rb, sg)
            scatter_rows(NSL * r + t, rb, ss)
        return carry

    lax.fori_loop(0, NR, edge_round, 0)
    for rb, sg, ss in slots:
        wait_scatter(rb, ss)

    @pl.when(w < REM)
    def _():
        ebase = cb + NJ * CH
        pltpu.sync_copy(src_hbm.at[pl.ds(ebase, CH)], idx_v)
        pltpu.sync_copy(dst_hbm.at[pl.ds(ebase, CH)], dst_v)
        for k in range(CH // 16):
            d16 = dst_v[pl.ds(k * 16, 16)]
            dst_v[pl.ds(k * 16, 16)] = plsc.load_gather(pos_v, [d16])
        pltpu.async_copy(y2_hbm.at[idx_v], rows_v, sg0).wait()
        pltpu.sync_copy(rows_v, agg_sp.at[dst_v], add=True)
        pltpu.sync_copy(ones_v, deg_sp.at[dst_v], add=True)

    plsc.subcore_barrier()

    def batch_half(h, carry):
        nbase = s * BPT + h * CH
        pltpu.sync_copy(nodes_hbm.at[pl.ds(nbase, CH)], idx_v)
        for k in range(CH // 16):
            n16 = idx_v[pl.ds(k * 16, 16)]
            dst_v[pl.ds(k * 16, 16)] = plsc.load_gather(pos_v, [n16])
        pltpu.async_copy(agg_sp.at[dst_v], rows_v, sem).wait()
        pltpu.sync_copy(rows_v, aggb_hbm.at[c, pl.ds(nbase, CH)])
        pltpu.async_copy(deg_sp.at[dst_v], deg1_v, sem).wait()
        pltpu.sync_copy(deg1_v, degb_hbm.at[c, pl.ds(nbase, CH)])
        return carry

    lax.fori_loop(0, BPT // CH, batch_half, 0)

    nb2 = w * CH
    pltpu.sync_copy(nodes_hbm.at[pl.ds(nb2, CH)], idx_v)
    pltpu.async_copy(z1_hbm.at[idx_v], rows_v, sem).wait()
    pltpu.sync_copy(rows_v, zb_hbm.at[pl.ds(nb2, CH)])


_sc_segment = functools.partial(
    pl.kernel,
    out_type=(
        jax.ShapeDtypeStruct((NC, B, H), jnp.float32),
        jax.ShapeDtypeStruct((NC, B), jnp.float32),
        jax.ShapeDtypeStruct((B, H), jnp.float32),
    ),
    mesh=plsc.VectorSubcoreMesh(
        core_axis_name="c", subcore_axis_name="s", num_cores=NC, num_subcores=NS
    ),
    compiler_params=pltpu.CompilerParams(needs_layout_passes=False),
    scratch_types=[
        pltpu.VMEM((CH,), jnp.int32),
        pltpu.VMEM((CH,), jnp.int32),
        pltpu.VMEM((CH,), jnp.int32),
        pltpu.VMEM((NPAD,), jnp.int32),
        pltpu.VMEM((CH, H), jnp.float32),
        pltpu.VMEM((CH, H), jnp.float32),
        pltpu.VMEM((CH, H), jnp.float32),
        pltpu.VMEM((CH,), jnp.float32),
        pltpu.VMEM((CH,), jnp.float32),
        pltpu.VMEM((CH,), jnp.float32),
        pltpu.VMEM((NJ * CH,), jnp.int32),
        pltpu.VMEM((NJ * CH,), jnp.int32),
        pltpu.VMEM((NJ, CH), jnp.int32),
        pltpu.VMEM_SHARED((BP, H), jnp.float32),
        pltpu.VMEM_SHARED((BP,), jnp.float32),
        pltpu.VMEM_SHARED((NPAD,), jnp.int32),
        pltpu.SemaphoreType.DMA,
        pltpu.SemaphoreType.DMA,
        pltpu.SemaphoreType.DMA,
        pltpu.SemaphoreType.DMA,
        pltpu.SemaphoreType.DMA,
        pltpu.SemaphoreType.DMA,
        pltpu.SemaphoreType.DMA,
    ],
)(_sc_body)


_mm_call = pl.pallas_call(
    _mm_body,
    grid=(N // MMB,),
    in_specs=[
        pl.BlockSpec((MMB, D), lambda i: (i, 0)),
        pl.BlockSpec((D, 2 * H), lambda i: (0, 0)),
    ],
    out_specs=[
        pl.BlockSpec((MMB, H), lambda i: (i, 0)),
        pl.BlockSpec((MMB, H), lambda i: (i, 0)),
    ],
    out_shape=[
        jax.ShapeDtypeStruct((N, H), jnp.float32),
        jax.ShapeDtypeStruct((N, H), jnp.float32),
    ],
)

_combine_call = pl.pallas_call(
    _combine_body,
    grid=(B // CB,),
    in_specs=[
        pl.BlockSpec((CB, H), lambda i: (i, 0)),
        pl.BlockSpec((CB, H), lambda i: (i, 0)),
        pl.BlockSpec((CB, H), lambda i: (i, 0)),
        pl.BlockSpec((CB,), lambda i: (i,)),
        pl.BlockSpec((CB,), lambda i: (i,)),
    ],
    out_specs=pl.BlockSpec((CB, H), lambda i: (i, 0)),
    out_shape=jax.ShapeDtypeStruct((B, H), jnp.float32),
)


def kernel(nodes, x, edge_index, W):
    wc = jnp.concatenate([W[:, :D].T, W[:, D:].T], axis=1)
    z1, y2 = _mm_call(x, wc)
    src = edge_index[0]
    dst = edge_index[1]
    aggb, degb, zb = _sc_segment(src, dst, nodes, y2, z1)
    return _combine_call(zb, aggb[0], aggb[1], degb[0], degb[1])


# speedup vs baseline: 1.2087x; 1.2087x over previous
import functools

import jax
import jax.numpy as jnp
from jax import lax
from jax.experimental import pallas as pl
from jax.experimental.pallas import tpu as pltpu
from jax.experimental.pallas import tpu_sc as plsc

N = 10000
E = 160000
D = 256
H = 128
B = 4096

NC = 2
NS = 16
NW = NC * NS
CH = 128
NCHUNK = E // CH
NJ = NCHUNK // NW
REM = NCHUNK % NW
NPAD = 10240
POS_PER_TILE = NPAD // NS
BP = 6144
DUMP = B
DSPREAD = 2048
APT = BP // NS
BPT = B // NS
MMB = 1000
CB = 1024


_DN = (((1,), (1,)), ((), ()))


def _mm_body(x_ref, w_ref, z1_ref, y2_ref):
    xb = x_ref[...]
    wb = w_ref[...]
    z1_ref[...] = lax.dot_general(xb, wb[:, :D], _DN,
                                  preferred_element_type=jnp.float32)
    y2_ref[...] = lax.dot_general(xb, wb[:, D:], _DN,
                                  preferred_element_type=jnp.float32)


def _combine_body(zb_ref, a0_ref, a1_ref, d0_ref, d1_ref, o_ref):
    deg = jnp.maximum(d0_ref[...] + d1_ref[...], 1.0)
    mean = (a0_ref[...] + a1_ref[...]) / deg[:, None]
    o_ref[...] = jnp.maximum(zb_ref[...] + mean, 0.0)


def _sc_body(src_hbm, dst_hbm, nodes_hbm, y2_hbm, z1_hbm,
             aggb_hbm, degb_hbm, zb_hbm,
             idx_v, dst_v, val_v, pos_v, rows_v, rows2_v, ones_v, zdeg_v,
             deg1_v, sbuf_v, dtmp_v, dbuf_v,
             agg_sp, deg_sp, pos_sp, sem, sg0, sg1, ss0, ss1):
    c = lax.axis_index("c")
    s = lax.axis_index("s")
    w = c * NS + s

    lane = lax.iota(jnp.int32, 16)
    zero16 = jnp.zeros((16,), jnp.float32)
    ones16 = jnp.ones((16,), jnp.float32)
    dump16 = jnp.full((16,), DUMP, jnp.int32)

    def fill_rows(i, carry):
        for j in range(H // 16):
            rows_v[i, pl.ds(j * 16, 16)] = zero16
        return carry

    lax.fori_loop(0, CH, fill_rows, 0)
    for j in range(CH // 16):
        ones_v[pl.ds(j * 16, 16)] = ones16
        zdeg_v[pl.ds(j * 16, 16)] = zero16

    def fill_dump(m, carry):
        g16 = s * POS_PER_TILE + m * 16 + lane
        sbuf_v[pl.ds(m * 16, 16)] = DUMP + (g16 & (DSPREAD - 1))
        return carry

    lax.fori_loop(0, POS_PER_TILE // 16, fill_dump, 0)
    zbase = s * APT
    pltpu.async_copy(sbuf_v.at[pl.ds(0, POS_PER_TILE)],
                     pos_sp.at[pl.ds(s * POS_PER_TILE, POS_PER_TILE)], sem)
    for m in range(APT // CH):
        pltpu.async_copy(rows_v, agg_sp.at[pl.ds(zbase + m * CH, CH)], sem)
        pltpu.async_copy(zdeg_v, deg_sp.at[pl.ds(zbase + m * CH, CH)], sem)
    pltpu.make_async_copy(sbuf_v.at[pl.ds(0, POS_PER_TILE)],
                          pos_sp.at[pl.ds(s * POS_PER_TILE, POS_PER_TILE)],
                          sem).wait()
    for m in range(APT // CH):
        pltpu.make_async_copy(rows_v, agg_sp.at[pl.ds(zbase + m * CH, CH)],
                              sem).wait()
        pltpu.make_async_copy(zdeg_v, deg_sp.at[pl.ds(zbase + m * CH, CH)],
                              sem).wait()
    plsc.subcore_barrier()

    def pos_scatter(h, carry):
        nbase = s * BPT + h * CH
        pltpu.sync_copy(nodes_hbm.at[pl.ds(nbase, CH)], idx_v)
        for k in range(CH // 16):
            val_v[pl.ds(k * 16, 16)] = nbase + k * 16 + lane
        pltpu.sync_copy(val_v, pos_sp.at[idx_v])
        return carry

    lax.fori_loop(0, BPT // CH, pos_scatter, 0)
    plsc.subcore_barrier()

    pltpu.sync_copy(pos_sp, pos_v)

    cb = (w * NJ + jnp.minimum(w, REM)) * CH
    pltpu.sync_copy(src_hbm.at[pl.ds(cb, NJ * CH)], sbuf_v)
    pltpu.sync_copy(dst_hbm.at[pl.ds(cb, NJ * CH)], dtmp_v)

    def gather_rows(la, rbuf, sg):
        pltpu.async_copy(y2_hbm.at[sbuf_v.at[pl.ds(la * CH, CH)]], rbuf, sg)

    def wait_gather(rbuf, sg):
        pltpu.make_async_copy(y2_hbm.at[sbuf_v.at[pl.ds(0, CH)]], rbuf, sg).wait()

    def scatter_rows(la, rbuf, ss):
        pltpu.async_copy(rbuf, agg_sp.at[dbuf_v.at[la]], ss, add=True)
        pltpu.async_copy(ones_v, deg_sp.at[dbuf_v.at[la]], ss, add=True)

    def wait_scatter(rbuf, ss):
        pltpu.make_async_copy(rbuf, agg_sp.at[dbuf_v.at[0]], ss).wait()
        pltpu.make_async_copy(ones_v, deg_sp.at[dbuf_v.at[0]], ss).wait()

    gather_rows(0, rows_v, sg0)

    def trans_body(j, carry):
        for k in range(CH // 16):
            d16 = dtmp_v[pl.ds(j * CH + k * 16, 16)]
            dbuf_v[j, pl.ds(k * 16, 16)] = plsc.load_gather(pos_v, [d16])
        return carry

    lax.fori_loop(0, NJ, trans_body, 0)

    NR = (NJ - 1) // 2

    def edge_round(r, carry):
        @pl.when(r > 0)
        def _():
            wait_scatter(rows2_v, ss1)

        gather_rows(2 * r + 1, rows2_v, sg1)
        wait_gather(rows_v, sg0)
        scatter_rows(2 * r, rows_v, ss0)

        @pl.when(r < NR - 1)
        def _():
            wait_scatter(rows_v, ss0)
            gather_rows(2 * r + 2, rows_v, sg0)

        wait_gather(rows2_v, sg1)
        scatter_rows(2 * r + 1, rows2_v, ss1)
        return carry

    lax.fori_loop(0, NR, edge_round, 0)
    wait_scatter(rows_v, ss0)
    wait_scatter(rows2_v, ss1)

    gather_rows(NJ - 1, rows_v, sg0)
    wait_gather(rows_v, sg0)
    scatter_rows(NJ - 1, rows_v, ss0)
    wait_scatter(rows_v, ss0)

    @pl.when(w < REM)
    def _():
        ebase = cb + NJ * CH
        pltpu.sync_copy(src_hbm.at[pl.ds(ebase, CH)], idx_v)
        pltpu.sync_copy(dst_hbm.at[pl.ds(ebase, CH)], dst_v)
        for k in range(CH // 16):
            d16 = dst_v[pl.ds(k * 16, 16)]
            dst_v[pl.ds(k * 16, 16)] = plsc.load_gather(pos_v, [d16])
        pltpu.async_copy(y2_hbm.at[idx_v], rows_v, sg0).wait()
        pltpu.sync_copy(rows_v, agg_sp.at[dst_v], add=True)
        pltpu.sync_copy(ones_v, deg_sp.at[dst_v], add=True)

    plsc.subcore_barrier()

    def batch_half(h, carry):
        nbase = s * BPT + h * CH
        pltpu.sync_copy(nodes_hbm.at[pl.ds(nbase, CH)], idx_v)
        for k in range(CH // 16):
            n16 = idx_v[pl.ds(k * 16, 16)]
            dst_v[pl.ds(k * 16, 16)] = plsc.load_gather(pos_v, [n16])
        pltpu.async_copy(agg_sp.at[dst_v], rows_v, sem).wait()
        pltpu.sync_copy(rows_v, aggb_hbm.at[c, pl.ds(nbase, CH)])
        pltpu.async_copy(deg_sp.at[dst_v], deg1_v, sem).wait()
        pltpu.sync_copy(deg1_v, degb_hbm.at[c, pl.ds(nbase, CH)])
        return carry

    lax.fori_loop(0, BPT // CH, batch_half, 0)

    nb2 = w * CH
    pltpu.sync_copy(nodes_hbm.at[pl.ds(nb2, CH)], idx_v)
    pltpu.async_copy(z1_hbm.at[idx_v], rows_v, sem).wait()
    pltpu.sync_copy(rows_v, zb_hbm.at[pl.ds(nb2, CH)])


_sc_segment = functools.partial(
    pl.kernel,
    out_type=(
        jax.ShapeDtypeStruct((NC, B, H), jnp.float32),
        jax.ShapeDtypeStruct((NC, B), jnp.float32),
        jax.ShapeDtypeStruct((B, H), jnp.float32),
    ),
    mesh=plsc.VectorSubcoreMesh(
        core_axis_name="c", subcore_axis_name="s", num_cores=NC, num_subcores=NS
    ),
    compiler_params=pltpu.CompilerParams(needs_layout_passes=False),
    scratch_types=[
        pltpu.VMEM((CH,), jnp.int32),
        pltpu.VMEM((CH,), jnp.int32),
        pltpu.VMEM((CH,), jnp.int32),
        pltpu.VMEM((NPAD,), jnp.int32),
        pltpu.VMEM((CH, H), jnp.float32),
        pltpu.VMEM((CH, H), jnp.float32),
        pltpu.VMEM((CH,), jnp.float32),
        pltpu.VMEM((CH,), jnp.float32),
        pltpu.VMEM((CH,), jnp.float32),
        pltpu.VMEM((NJ * CH,), jnp.int32),
        pltpu.VMEM((NJ * CH,), jnp.int32),
        pltpu.VMEM((NJ, CH), jnp.int32),
        pltpu.VMEM_SHARED((BP, H), jnp.float32),
        pltpu.VMEM_SHARED((BP,), jnp.float32),
        pltpu.VMEM_SHARED((NPAD,), jnp.int32),
        pltpu.SemaphoreType.DMA,
        pltpu.SemaphoreType.DMA,
        pltpu.SemaphoreType.DMA,
        pltpu.SemaphoreType.DMA,
        pltpu.SemaphoreType.DMA,
    ],
)(_sc_body)


_mm_call = pl.pallas_call(
    _mm_body,
    grid=(N // MMB,),
    in_specs=[
        pl.BlockSpec((MMB, D), lambda i: (i, 0)),
        pl.BlockSpec((H, 2 * D), lambda i: (0, 0)),
    ],
    out_specs=[
        pl.BlockSpec((MMB, H), lambda i: (i, 0)),
        pl.BlockSpec((MMB, H), lambda i: (i, 0)),
    ],
    out_shape=[
        jax.ShapeDtypeStruct((N, H), jnp.float32),
        jax.ShapeDtypeStruct((N, H), jnp.float32),
    ],
)

_combine_call = pl.pallas_call(
    _combine_body,
    grid=(B // CB,),
    in_specs=[
        pl.BlockSpec((CB, H), lambda i: (i, 0)),
        pl.BlockSpec((CB, H), lambda i: (i, 0)),
        pl.BlockSpec((CB, H), lambda i: (i, 0)),
        pl.BlockSpec((CB,), lambda i: (i,)),
        pl.BlockSpec((CB,), lambda i: (i,)),
    ],
    out_specs=pl.BlockSpec((CB, H), lambda i: (i, 0)),
    out_shape=jax.ShapeDtypeStruct((B, H), jnp.float32),
)


def kernel(nodes, x, edge_index, W):
    z1, y2 = _mm_call(x, W)
    src = edge_index[0]
    dst = edge_index[1]
    aggb, degb, zb = _sc_segment(src, dst, nodes, y2, z1)
    return _combine_call(zb, aggb[0], aggb[1], degb[0], degb[1])


# speedup vs baseline: 1.2239x; 1.0125x over previous
import functools

import jax
import jax.numpy as jnp
from jax import lax
from jax.experimental import pallas as pl
from jax.experimental.pallas import tpu as pltpu
from jax.experimental.pallas import tpu_sc as plsc

N = 10000
E = 160000
D = 256
H = 128
B = 4096

NC = 2
NS = 16
NW = NC * NS
CH = 128
NCHUNK = E // CH
NJ = NCHUNK // NW
REM = NCHUNK % NW
NPAD = 10240
POS_PER_TILE = NPAD // NS
BP = 6144
DUMP = B
DSPREAD = 2048
APT = BP // NS
BPT = B // NS
MMB = 1000
CB = 1024


_DN = (((1,), (1,)), ((), ()))


def _mm_body(x_ref, w_ref, z1_ref, y2_ref):
    xb = x_ref[...]
    wb = w_ref[...]
    z1_ref[...] = lax.dot_general(xb, wb[:, :D], _DN,
                                  preferred_element_type=jnp.float32)
    y2_ref[...] = lax.dot_general(xb, wb[:, D:], _DN,
                                  preferred_element_type=jnp.float32)


def _combine_body(zb_ref, a0_ref, a1_ref, d0_ref, d1_ref, o_ref):
    deg = jnp.maximum(d0_ref[...] + d1_ref[...], 1.0)
    mean = (a0_ref[...] + a1_ref[...]) / deg[:, None]
    o_ref[...] = jnp.maximum(zb_ref[...] + mean, 0.0)


def _sc_body(src_hbm, dst_hbm, nodes_hbm, y2_hbm, z1_hbm,
             aggb_hbm, degb_hbm, zb_hbm,
             idx_v, dst_v, val_v, pos_v, rows_v, rows2_v, ones_v, zdeg_v,
             deg1_v, sbuf_v, dtmp_v, dbuf_v,
             agg_sp, deg_sp, pos_sp, sem, sg0, sg1, ss0, ss1):
    c = lax.axis_index("c")
    s = lax.axis_index("s")
    w = c * NS + s

    lane = lax.iota(jnp.int32, 16)
    zero16 = jnp.zeros((16,), jnp.float32)
    ones16 = jnp.ones((16,), jnp.float32)
    dump16 = jnp.full((16,), DUMP, jnp.int32)

    def fill_rows(i, carry):
        for j in range(H // 16):
            rows_v[i, pl.ds(j * 16, 16)] = zero16
        return carry

    lax.fori_loop(0, CH, fill_rows, 0)
    for j in range(CH // 16):
        ones_v[pl.ds(j * 16, 16)] = ones16
        zdeg_v[pl.ds(j * 16, 16)] = zero16

    def fill_dump(m, carry):
        g16 = s * POS_PER_TILE + m * 16 + lane
        sbuf_v[pl.ds(m * 16, 16)] = DUMP + (g16 & (DSPREAD - 1))
        return carry

    lax.fori_loop(0, POS_PER_TILE // 16, fill_dump, 0)
    zbase = s * APT
    pltpu.async_copy(sbuf_v.at[pl.ds(0, POS_PER_TILE)],
                     pos_sp.at[pl.ds(s * POS_PER_TILE, POS_PER_TILE)], sem)
    for m in range(APT // CH):
        pltpu.async_copy(rows_v, agg_sp.at[pl.ds(zbase + m * CH, CH)], sem)
        pltpu.async_copy(zdeg_v, deg_sp.at[pl.ds(zbase + m * CH, CH)], sem)
    pltpu.make_async_copy(sbuf_v.at[pl.ds(0, POS_PER_TILE)],
                          pos_sp.at[pl.ds(s * POS_PER_TILE, POS_PER_TILE)],
                          sem).wait()
    for m in range(APT // CH):
        pltpu.make_async_copy(rows_v, agg_sp.at[pl.ds(zbase + m * CH, CH)],
                              sem).wait()
        pltpu.make_async_copy(zdeg_v, deg_sp.at[pl.ds(zbase + m * CH, CH)],
                              sem).wait()
    plsc.subcore_barrier()

    def pos_scatter(h, carry):
        nbase = s * BPT + h * CH
        pltpu.sync_copy(nodes_hbm.at[pl.ds(nbase, CH)], idx_v)
        for k in range(CH // 16):
            val_v[pl.ds(k * 16, 16)] = nbase + k * 16 + lane
        pltpu.sync_copy(val_v, pos_sp.at[idx_v])
        return carry

    lax.fori_loop(0, BPT // CH, pos_scatter, 0)
    plsc.subcore_barrier()

    pltpu.sync_copy(pos_sp, pos_v)

    cb = (w * NJ + jnp.minimum(w, REM)) * CH
    pltpu.sync_copy(src_hbm.at[pl.ds(cb, NJ * CH)], sbuf_v)
    pltpu.sync_copy(dst_hbm.at[pl.ds(cb, NJ * CH)], dtmp_v)

    def gather_rows(la, rbuf, sg):
        pltpu.async_copy(y2_hbm.at[sbuf_v.at[pl.ds(la * CH, CH)]], rbuf, sg)

    def wait_gather(rbuf, sg):
        pltpu.make_async_copy(y2_hbm.at[sbuf_v.at[pl.ds(0, CH)]], rbuf, sg).wait()

    def scatter_rows(la, rbuf, ss):
        pltpu.async_copy(rbuf, agg_sp.at[dbuf_v.at[la]], ss, add=True)
        pltpu.async_copy(ones_v, deg_sp.at[dbuf_v.at[la]], ss, add=True)

    def wait_scatter(rbuf, ss):
        pltpu.make_async_copy(rbuf, agg_sp.at[dbuf_v.at[0]], ss).wait()
        pltpu.make_async_copy(ones_v, deg_sp.at[dbuf_v.at[0]], ss).wait()

    gather_rows(0, rows_v, sg0)

    def trans_body(j, carry):
        for k in range(CH // 16):
            d16 = dtmp_v[pl.ds(j * CH + k * 16, 16)]
            dbuf_v[j, pl.ds(k * 16, 16)] = plsc.load_gather(pos_v, [d16])
        return carry

    lax.fori_loop(0, NJ, trans_body, 0)

    NR = (NJ - 1) // 2

    def edge_round(r, carry):
        @pl.when(r > 0)
        def _():
            wait_scatter(rows2_v, ss1)

        gather_rows(2 * r + 1, rows2_v, sg1)
        wait_gather(rows_v, sg0)
        scatter_rows(2 * r, rows_v, ss0)

        @pl.when(r < NR - 1)
        def _():
            wait_scatter(rows_v, ss0)
            gather_rows(2 * r + 2, rows_v, sg0)

        wait_gather(rows2_v, sg1)
        scatter_rows(2 * r + 1, rows2_v, ss1)
        return carry

    lax.fori_loop(0, NR, edge_round, 0)
    wait_scatter(rows_v, ss0)
    wait_scatter(rows2_v, ss1)

    gather_rows(NJ - 1, rows_v, sg0)
    wait_gather(rows_v, sg0)
    scatter_rows(NJ - 1, rows_v, ss0)
    wait_scatter(rows_v, ss0)

    @pl.when(w < REM)
    def _():
        ebase = cb + NJ * CH
        pltpu.sync_copy(src_hbm.at[pl.ds(ebase, CH)], idx_v)
        pltpu.sync_copy(dst_hbm.at[pl.ds(ebase, CH)], dst_v)
        for k in range(CH // 16):
            d16 = dst_v[pl.ds(k * 16, 16)]
            dst_v[pl.ds(k * 16, 16)] = plsc.load_gather(pos_v, [d16])
        pltpu.async_copy(y2_hbm.at[idx_v], rows_v, sg0).wait()
        pltpu.sync_copy(rows_v, agg_sp.at[dst_v], add=True)
        pltpu.sync_copy(ones_v, deg_sp.at[dst_v], add=True)

    plsc.subcore_barrier()

    nbase = s * BPT
    nb2 = w * CH
    pltpu.async_copy(nodes_hbm.at[pl.ds(nbase, BPT)], sbuf_v.at[pl.ds(0, BPT)],
                     sem)
    pltpu.sync_copy(nodes_hbm.at[pl.ds(nb2, CH)], idx_v)
    pltpu.make_async_copy(nodes_hbm.at[pl.ds(nbase, BPT)],
                          sbuf_v.at[pl.ds(0, BPT)], sem).wait()
    for h in range(BPT // CH):
        for k in range(CH // 16):
            n16 = sbuf_v[pl.ds(h * CH + k * 16, 16)]
            dbuf_v[h, pl.ds(k * 16, 16)] = plsc.load_gather(pos_v, [n16])
    pltpu.async_copy(agg_sp.at[dbuf_v.at[0]], rows_v, sg0)
    pltpu.async_copy(agg_sp.at[dbuf_v.at[1]], rows2_v, sg1)
    pltpu.async_copy(deg_sp.at[dbuf_v.at[0]], deg1_v, ss0)
    pltpu.async_copy(deg_sp.at[dbuf_v.at[1]], zdeg_v, ss1)
    pltpu.make_async_copy(agg_sp.at[dbuf_v.at[0]], rows_v, sg0).wait()
    pltpu.sync_copy(rows_v, aggb_hbm.at[c, pl.ds(nbase, CH)])
    pltpu.async_copy(z1_hbm.at[idx_v], rows_v, sg0)
    pltpu.make_async_copy(agg_sp.at[dbuf_v.at[1]], rows2_v, sg1).wait()
    pltpu.sync_copy(rows2_v, aggb_hbm.at[c, pl.ds(nbase + CH, CH)])
    pltpu.make_async_copy(deg_sp.at[dbuf_v.at[0]], deg1_v, ss0).wait()
    pltpu.sync_copy(deg1_v, degb_hbm.at[c, pl.ds(nbase, CH)])
    pltpu.make_async_copy(deg_sp.at[dbuf_v.at[1]], zdeg_v, ss1).wait()
    pltpu.sync_copy(zdeg_v, degb_hbm.at[c, pl.ds(nbase + CH, CH)])
    pltpu.make_async_copy(z1_hbm.at[idx_v], rows_v, sg0).wait()
    pltpu.sync_copy(rows_v, zb_hbm.at[pl.ds(nb2, CH)])


_sc_segment = functools.partial(
    pl.kernel,
    out_type=(
        jax.ShapeDtypeStruct((NC, B, H), jnp.float32),
        jax.ShapeDtypeStruct((NC, B), jnp.float32),
        jax.ShapeDtypeStruct((B, H), jnp.float32),
    ),
    mesh=plsc.VectorSubcoreMesh(
        core_axis_name="c", subcore_axis_name="s", num_cores=NC, num_subcores=NS
    ),
    compiler_params=pltpu.CompilerParams(needs_layout_passes=False),
    scratch_types=[
        pltpu.VMEM((CH,), jnp.int32),
        pltpu.VMEM((CH,), jnp.int32),
        pltpu.VMEM((CH,), jnp.int32),
        pltpu.VMEM((NPAD,), jnp.int32),
        pltpu.VMEM((CH, H), jnp.float32),
        pltpu.VMEM((CH, H), jnp.float32),
        pltpu.VMEM((CH,), jnp.float32),
        pltpu.VMEM((CH,), jnp.float32),
        pltpu.VMEM((CH,), jnp.float32),
        pltpu.VMEM((NJ * CH,), jnp.int32),
        pltpu.VMEM((NJ * CH,), jnp.int32),
        pltpu.VMEM((NJ, CH), jnp.int32),
        pltpu.VMEM_SHARED((BP, H), jnp.float32),
        pltpu.VMEM_SHARED((BP,), jnp.float32),
        pltpu.VMEM_SHARED((NPAD,), jnp.int32),
        pltpu.SemaphoreType.DMA,
        pltpu.SemaphoreType.DMA,
        pltpu.SemaphoreType.DMA,
        pltpu.SemaphoreType.DMA,
        pltpu.SemaphoreType.DMA,
    ],
)(_sc_body)


_mm_call = pl.pallas_call(
    _mm_body,
    grid=(N // MMB,),
    in_specs=[
        pl.BlockSpec((MMB, D), lambda i: (i, 0)),
        pl.BlockSpec((H, 2 * D), lambda i: (0, 0)),
    ],
    out_specs=[
        pl.BlockSpec((MMB, H), lambda i: (i, 0)),
        pl.BlockSpec((MMB, H), lambda i: (i, 0)),
    ],
    out_shape=[
        jax.ShapeDtypeStruct((N, H), jnp.float32),
        jax.ShapeDtypeStruct((N, H), jnp.float32),
    ],
)

_combine_call = pl.pallas_call(
    _combine_body,
    grid=(B // CB,),
    in_specs=[
        pl.BlockSpec((CB, H), lambda i: (i, 0)),
        pl.BlockSpec((CB, H), lambda i: (i, 0)),
        pl.BlockSpec((CB, H), lambda i: (i, 0)),
        pl.BlockSpec((CB,), lambda i: (i,)),
        pl.BlockSpec((CB,), lambda i: (i,)),
    ],
    out_specs=pl.BlockSpec((CB, H), lambda i: (i, 0)),
    out_shape=jax.ShapeDtypeStruct((B, H), jnp.float32),
)


def kernel(nodes, x, edge_index, W):
    z1, y2 = _mm_call(x, W)
    src = edge_index[0]
    dst = edge_index[1]
    aggb, degb, zb = _sc_segment(src, dst, nodes, y2, z1)
    return _combine_call(zb, aggb[0], aggb[1], degb[0], degb[1])
